# Initial kernel scaffold; baseline (speedup 1.0000x reference)
import jax
import jax.numpy as jnp
from jax.experimental import pallas as pl


def kernel(x, W1, b1, W2, b2, W3, b3, We, be, Wd, bd):
    raise NotImplementedError("write your pallas kernel here")


# speedup vs baseline: 48.8756x; 48.8756x over previous
import functools
import numpy as np

import jax
import jax.numpy as jnp
from jax.experimental import pallas as pl
from jax.experimental.pallas import tpu as pltpu

N = 256
DIM_H = 16
M_DIM = 32
CUTOFF = 3.6
LANES = 128


def _atan(x):
    t = jnp.abs(x)
    c1 = t > 2.414213562373095
    c2 = t > 0.4142135623730951
    base = jnp.where(c1, np.float32(np.pi / 2),
                     jnp.where(c2, np.float32(np.pi / 4), np.float32(0.0)))
    arg = jnp.where(c1, -1.0 / t, jnp.where(c2, (t - 1.0) / (t + 1.0), t))
    z = arg * arg
    p = (((8.05374449538e-2 * z - 1.38776856032e-1) * z
          + 1.99777106478e-1) * z - 3.33329491539e-1) * z * arg + arg
    return jnp.sign(x) * (base + p)


def _tc_body(xp_ref, xt_ref, w1_ref, b1_ref, w2_ref, b2_ref, w3_ref, b3_ref,
             we_ref, be_ref, wd_ref, bd_ref, out_ref):
    f32 = jnp.float32
    xp = xp_ref[...]
    lane = jax.lax.broadcasted_iota(jnp.int32, (N, LANES), 1)

    acc = jnp.zeros((N, N), f32)
    for d in range(3):
        col = jnp.sum(jnp.where(lane == d, xp, 0.0), axis=1, keepdims=True)
        row = xt_ref[d:d + 1, :]
        acc = acc + jnp.abs(col - row)
    graph = (acc <= CUTOFF).astype(f32)
    ri = jax.lax.broadcasted_iota(jnp.int32, (N, N), 0)
    ci = jax.lax.broadcasted_iota(jnp.int32, (N, N), 1)
    tri = (ri >= ci).astype(f32)
    cntm = jax.lax.dot(tri, graph, preferred_element_type=f32)
    mask = (cntm > 0.0).astype(f32)

    h = _atan(jax.lax.dot(xp, w1_ref[...], preferred_element_type=f32)
              + b1_ref[0:1, :])
    h = _atan(jax.lax.dot(h, w2_ref[...], preferred_element_type=f32)
              + b2_ref[0:1, :])
    m = jax.lax.dot(h, w3_ref[...], preferred_element_type=f32) + b3_ref[0:1, :]

    ki = jax.lax.broadcasted_iota(jnp.int32, (LANES, LANES), 0)
    kj = jax.lax.broadcasted_iota(jnp.int32, (LANES, LANES), 1)
    perm = ((ki == kj + 8) & (kj >= DIM_H) & (kj < DIM_H + 8)).astype(f32)
    mq = jax.lax.dot(m, perm, preferred_element_type=f32)
    prod = m * mq
    a = jnp.sum(jnp.where((lane >= DIM_H) & (lane < DIM_H + 8), prod, 0.0),
                axis=1, keepdims=True)
    amax = jnp.maximum(jnp.max(a), 0.0)
    e = jnp.exp(a - amax)

    x_mat = (jnp.where(lane < DIM_H, m, 0.0) * e
             + jnp.where(lane == DIM_H, e, 0.0)
             + jnp.where(lane == DIM_H + 1, 1.0, 0.0))
    s = jax.lax.dot(mask, x_mat, preferred_element_type=f32)
    pe = jnp.sum(jnp.where(lane == DIM_H, s, 0.0), axis=1, keepdims=True)
    cnt = jnp.sum(jnp.where(lane == DIM_H + 1, s, 0.0), axis=1, keepdims=True)
    den = pe + (np.float32(N) - cnt) * jnp.exp(-amax)
    my = jnp.where(lane < DIM_H, s, 0.0) / den

    code = _atan(jax.lax.dot(my, we_ref[...], preferred_element_type=f32)
                 + be_ref[0:1, :])
    out_ref[...] = (jax.lax.dot(code, wd_ref[...], preferred_element_type=f32)
                    + bd_ref[0:1, :])


def _pad2(a, rows, cols):
    return jnp.zeros((rows, cols), jnp.float32).at[:a.shape[0], :a.shape[1]].set(a)


def _pad_bias(b):
    return jnp.tile(jnp.zeros((1, LANES), jnp.float32).at[0, :b.shape[0]].set(b),
                    (8, 1))


@jax.jit
def kernel(x, W1, b1, W2, b2, W3, b3, We, be, Wd, bd):
    xp = _pad2(x, N, LANES)
    xt = _pad2(x[:, 0:3].T, 8, N)
    args = (
        xp, xt,
        _pad2(W1.T, LANES, LANES), _pad_bias(b1),
        _pad2(W2.T, LANES, LANES), _pad_bias(b2),
        _pad2(W3.T, LANES, LANES), _pad_bias(b3),
        _pad2(We.T, LANES, LANES), _pad_bias(be),
        _pad2(Wd.T, LANES, LANES), _pad_bias(bd),
    )
    out = pl.pallas_call(
        _tc_body,
        out_shape=jax.ShapeDtypeStruct((N, LANES), jnp.float32),
    )(*args)
    return out[:, :7]
